# Initial kernel scaffold; baseline (speedup 1.0000x reference)
import functools

import jax
import jax.numpy as jnp
from jax.experimental import pallas as pl
from jax.experimental.pallas import tpu as pltpu
from jax.experimental.pallas import tpu_sc as plsc

_WINDOW = 64


@functools.lru_cache(maxsize=None)
def _make_gather(num_indices: int, dim: int, dtype):
    mesh = plsc.VectorSubcoreMesh(
        core_axis_name="core", subcore_axis_name="subcore"
    )

    @functools.partial(
        pl.kernel,
        out_type=jax.ShapeDtypeStruct((num_indices, dim), dtype),
        mesh=mesh,
    )
    def gather_kernel(table_hbm, ids_hbm, out_hbm):
        def body(i_vmem, o_vmem):
            pltpu.sync_copy(table_hbm.at[i_vmem.at[0]], o_vmem)

        pltpu.emit_pipeline(
            body,
            grid=(num_indices // _WINDOW,),
            in_specs=[
                pl.BlockSpec((1, _WINDOW), index_map=lambda i: (0, i))
            ],
            out_specs=[
                pl.BlockSpec((_WINDOW, dim), index_map=lambda i: (i, 0))
            ],
            core_axis_name=("core", "subcore"),
            dimension_semantics=(pltpu.PARALLEL,),
        )(ids_hbm, out_hbm)

    return gather_kernel


def kernel(token_ids, embedding_table):
    b, s = token_ids.shape
    _, d = embedding_table.shape
    n = b * s
    ids = token_ids.reshape(1, n).astype(jnp.int32)
    out = _make_gather(n, d, embedding_table.dtype)(embedding_table, ids)
    return out.reshape(b, s, d)


# speedup vs baseline: 1.8191x; 1.8191x over previous
import functools

import jax
import jax.numpy as jnp
from jax import lax
from jax.experimental import pallas as pl
from jax.experimental.pallas import tpu as pltpu
from jax.experimental.pallas import tpu_sc as plsc

_CHUNK = 64


@functools.lru_cache(maxsize=None)
def _make_gather(num_indices: int, dim: int, dtype):
    info = plsc.get_sparse_core_info()
    nw = info.num_cores * info.num_subcores
    assert num_indices % (nw * 2 * _CHUNK) == 0
    per_w = num_indices // nw
    nchunk = per_w // _CHUNK

    mesh = plsc.VectorSubcoreMesh(
        core_axis_name="core", subcore_axis_name="subcore"
    )

    @functools.partial(
        pl.kernel,
        out_type=jax.ShapeDtypeStruct((num_indices, dim), dtype),
        mesh=mesh,
        scratch_types=[
            pltpu.VMEM((per_w,), jnp.int32),
            pltpu.VMEM((_CHUNK, dim), dtype),
            pltpu.VMEM((_CHUNK, dim), dtype),
            pltpu.SemaphoreType.DMA,
            pltpu.SemaphoreType.DMA,
            pltpu.SemaphoreType.DMA,
            pltpu.SemaphoreType.DMA,
        ],
    )
    def gather_kernel(table_hbm, ids_hbm, out_hbm, idx_v, buf0, buf1,
                      sg0, sg1, ss0, ss1):
        wid = lax.axis_index("subcore") * info.num_cores + lax.axis_index("core")
        base = wid * per_w
        pltpu.sync_copy(ids_hbm.at[pl.ds(base, per_w)], idx_v)

        @pl.loop(0, nchunk, step=2)
        def _(c):
            r0 = base + c * _CHUNK
            r1 = base + (c + 1) * _CHUNK
            g0 = pltpu.async_copy(
                table_hbm.at[idx_v.at[pl.ds(c * _CHUNK, _CHUNK)]], buf0, sg0)
            g1 = pltpu.async_copy(
                table_hbm.at[idx_v.at[pl.ds((c + 1) * _CHUNK, _CHUNK)]],
                buf1, sg1)
            g0.wait()
            s0 = pltpu.async_copy(buf0, out_hbm.at[pl.ds(r0, _CHUNK)], ss0)
            g1.wait()
            s1 = pltpu.async_copy(buf1, out_hbm.at[pl.ds(r1, _CHUNK)], ss1)
            s0.wait()
            s1.wait()

    return gather_kernel


def kernel(token_ids, embedding_table):
    b, s = token_ids.shape
    _, d = embedding_table.shape
    n = b * s
    ids = token_ids.reshape(n).astype(jnp.int32)
    out = _make_gather(n, d, embedding_table.dtype)(embedding_table, ids)
    return out.reshape(b, s, d)


# speedup vs baseline: 1.8553x; 1.0199x over previous
import functools

import jax
import jax.numpy as jnp
from jax import lax
from jax.experimental import pallas as pl
from jax.experimental.pallas import tpu as pltpu
from jax.experimental.pallas import tpu_sc as plsc

_CHUNK = 32
_NBUF = 4


@functools.lru_cache(maxsize=None)
def _make_gather(num_indices: int, dim: int, dtype):
    info = plsc.get_sparse_core_info()
    nw = info.num_cores * info.num_subcores
    per_w = num_indices // nw
    nchunk = per_w // _CHUNK
    assert num_indices % (nw * _CHUNK) == 0
    assert nchunk % _NBUF == 0 and nchunk >= 2 * _NBUF

    mesh = plsc.VectorSubcoreMesh(
        core_axis_name="core", subcore_axis_name="subcore"
    )

    @functools.partial(
        pl.kernel,
        out_type=jax.ShapeDtypeStruct((num_indices, dim), dtype),
        mesh=mesh,
        scratch_types=[
            pltpu.VMEM((per_w,), jnp.int32),
        ]
        + [pltpu.VMEM((_CHUNK, dim), dtype) for _ in range(_NBUF)]
        + [pltpu.SemaphoreType.DMA for _ in range(2 * _NBUF)],
    )
    def gather_kernel(table_hbm, ids_hbm, out_hbm, idx_v, *rest):
        bufs = rest[:_NBUF]
        sg = rest[_NBUF:2 * _NBUF]
        ss = rest[2 * _NBUF:3 * _NBUF]

        wid = (lax.axis_index("subcore") * info.num_cores
               + lax.axis_index("core"))
        base = wid * per_w
        pltpu.sync_copy(ids_hbm.at[pl.ds(base, per_w)], idx_v)

        def issue_gather(cc, b):
            pltpu.async_copy(
                table_hbm.at[idx_v.at[pl.ds(cc * _CHUNK, _CHUNK)]],
                bufs[b], sg[b])

        def wait_gather(b):
            pltpu.make_async_copy(
                table_hbm.at[pl.ds(0, _CHUNK)], bufs[b], sg[b]).wait()

        def issue_store(cc, b):
            pltpu.async_copy(
                bufs[b], out_hbm.at[pl.ds(base + cc * _CHUNK, _CHUNK)],
                ss[b])

        def wait_store(b):
            pltpu.make_async_copy(
                bufs[b], out_hbm.at[pl.ds(0, _CHUNK)], ss[b]).wait()

        issue_gather(0, 0)
        issue_gather(1, 1)
        for b in range(_NBUF):
            bn = (b + 2) % _NBUF
            if b >= 2:
                wait_store(bn)
            issue_gather(b + 2, bn)
            wait_gather(b)
            issue_store(b, b)

        @pl.loop(_NBUF, nchunk - _NBUF, step=_NBUF)
        def _(c):
            for b in range(_NBUF):
                cc = c + b
                bn = (b + 2) % _NBUF
                wait_store(bn)
                issue_gather(cc + 2, bn)
                wait_gather(b)
                issue_store(cc, b)

        for b in range(_NBUF):
            cc = nchunk - _NBUF + b
            bn = (b + 2) % _NBUF
            if b < 2:
                wait_store(bn)
                issue_gather(cc + 2, bn)
            wait_gather(b)
            issue_store(cc, b)
        for b in range(_NBUF):
            wait_store(b)

    return gather_kernel


def kernel(token_ids, embedding_table):
    b, s = token_ids.shape
    _, d = embedding_table.shape
    n = b * s
    ids = token_ids.reshape(n).astype(jnp.int32)
    out = _make_gather(n, d, embedding_table.dtype)(embedding_table, ids)
    return out.reshape(b, s, d)


# speedup vs baseline: 1.8577x; 1.0013x over previous
import functools

import jax
import jax.numpy as jnp
from jax import lax
from jax.experimental import pallas as pl
from jax.experimental.pallas import tpu as pltpu
from jax.experimental.pallas import tpu_sc as plsc

_CHUNK = 16
_NBUF = 8


@functools.lru_cache(maxsize=None)
def _make_gather(num_indices: int, dim: int, dtype):
    info = plsc.get_sparse_core_info()
    nw = info.num_cores * info.num_subcores
    per_w = num_indices // nw
    nchunk = per_w // _CHUNK
    look = _NBUF // 2
    assert num_indices % (nw * _CHUNK) == 0
    assert nchunk % _NBUF == 0 and nchunk >= 2 * _NBUF

    mesh = plsc.VectorSubcoreMesh(
        core_axis_name="core", subcore_axis_name="subcore"
    )

    @functools.partial(
        pl.kernel,
        out_type=jax.ShapeDtypeStruct((num_indices, dim), dtype),
        mesh=mesh,
        scratch_types=[
            pltpu.VMEM((per_w,), jnp.int32),
        ]
        + [pltpu.VMEM((_CHUNK, dim), dtype) for _ in range(_NBUF)]
        + [pltpu.SemaphoreType.DMA for _ in range(2 * _NBUF)],
    )
    def gather_kernel(table_hbm, ids_hbm, out_hbm, idx_v, *rest):
        bufs = rest[:_NBUF]
        sg = rest[_NBUF:2 * _NBUF]
        ss = rest[2 * _NBUF:3 * _NBUF]

        wid = (lax.axis_index("subcore") * info.num_cores
               + lax.axis_index("core"))
        base = wid * per_w
        pltpu.sync_copy(ids_hbm.at[pl.ds(base, per_w)], idx_v)

        def issue_gather(cc, b):
            pltpu.async_copy(
                table_hbm.at[idx_v.at[pl.ds(cc * _CHUNK, _CHUNK)]],
                bufs[b], sg[b])

        def wait_gather(b):
            pltpu.make_async_copy(
                table_hbm.at[pl.ds(0, _CHUNK)], bufs[b], sg[b]).wait()

        def issue_store(cc, b):
            pltpu.async_copy(
                bufs[b], out_hbm.at[pl.ds(base + cc * _CHUNK, _CHUNK)],
                ss[b])

        def wait_store(b):
            pltpu.make_async_copy(
                bufs[b], out_hbm.at[pl.ds(0, _CHUNK)], ss[b]).wait()

        def visit(cc, b, prefetch=True, free=True):
            bn = (b + look) % _NBUF
            if free:
                wait_store(bn)
            if prefetch:
                issue_gather(cc + look, bn)
            wait_gather(b)
            issue_store(cc, b)

        for b in range(look):
            issue_gather(b, b)
        for b in range(_NBUF):
            visit(b, b, free=(b >= look))

        @pl.loop(_NBUF, nchunk - _NBUF, step=_NBUF)
        def _(c):
            for b in range(_NBUF):
                visit(c + b, b)

        for b in range(_NBUF):
            visit(nchunk - _NBUF + b, b, prefetch=(b < look),
                  free=(b < look))
        for b in range(_NBUF):
            wait_store(b)

    return gather_kernel


def kernel(token_ids, embedding_table):
    b, s = token_ids.shape
    _, d = embedding_table.shape
    n = b * s
    ids = token_ids.reshape(n).astype(jnp.int32)
    out = _make_gather(n, d, embedding_table.dtype)(embedding_table, ids)
    return out.reshape(b, s, d)
